# baseline (speedup 1.0000x reference)
import functools

import jax
import jax.numpy as jnp
from jax import lax
from jax.experimental import pallas as pl
from jax.experimental.pallas import tpu as pltpu
from jax.experimental.pallas import tpu_sc as plsc

N = 10000
D = 128
E = 320000
NC = 2
NS = 16
NW = NC * NS
CH = 128
NCHUNK = 80
EPW = NCHUNK * CH
EPAD = EPW * NW
NP = 10240
RPT = NP // NS
_F32 = jnp.float32

_MESH = plsc.VectorSubcoreMesh(core_axis_name="c", subcore_axis_name="s")


@functools.partial(
    pl.kernel,
    out_type=jax.ShapeDtypeStruct((NC, NP, 16), _F32),
    mesh=_MESH,
    scratch_types=[
        pltpu.VMEM((NCHUNK, 2, CH), jnp.int32),
        pltpu.VMEM((CH, 16), _F32),
        pltpu.VMEM((16, 16), _F32),
        pltpu.VMEM_SHARED((NP, 16), _F32),
        pltpu.SemaphoreType.DMA,
    ],
)
def _sc_degree(edg_hbm, out_hbm, idx_v, ones_v, zbuf, hist, sem):
    c = lax.axis_index("c")
    s = lax.axis_index("s")
    wid = s * NC + c
    pltpu.async_copy(edg_hbm.at[wid], idx_v, sem)
    for r in range(16):
        zbuf[r] = jnp.zeros((16,), _F32)
    for r in range(CH):
        ones_v[r] = jnp.ones((16,), _F32)
    row0 = s * RPT
    def _zero(i, carry):
        pltpu.sync_copy(zbuf, hist.at[pl.ds(row0 + i * 16, 16), :])
        return carry
    lax.fori_loop(0, RPT // 16, _zero, 0)
    pltpu.make_async_copy(edg_hbm.at[wid], idx_v, sem).wait()
    plsc.subcore_barrier()
    def _acc(j, carry):
        pltpu.sync_copy(ones_v, hist.at[idx_v.at[j, 1]], add=True)
        return carry
    lax.fori_loop(0, NCHUNK, _acc, 0)
    plsc.subcore_barrier()
    pltpu.sync_copy(hist.at[pl.ds(row0, RPT), :],
                    out_hbm.at[c, pl.ds(row0, RPT), :])


@functools.partial(
    pl.kernel,
    out_type=jax.ShapeDtypeStruct((NC, NP, D), _F32),
    mesh=_MESH,
    scratch_types=[
        pltpu.VMEM((2, CH), jnp.int32),
        pltpu.VMEM((2, CH), jnp.int32),
        pltpu.VMEM((2, CH), jnp.int32),
        pltpu.VMEM((2, CH), jnp.int32),
        pltpu.VMEM((CH, D), _F32),
        pltpu.VMEM((CH, D), _F32),
        pltpu.VMEM_SHARED((NP, D), _F32),
        pltpu.SemaphoreType.DMA,
        pltpu.SemaphoreType.DMA,
        pltpu.SemaphoreType.DMA,
        pltpu.SemaphoreType.DMA,
        pltpu.SemaphoreType.DMA,
        pltpu.SemaphoreType.DMA,
        pltpu.SemaphoreType.DMA,
        pltpu.SemaphoreType.DMA,
    ],
)
def _sc_aggregate(hs_hbm, edg_hbm, out_hbm,
                  ia0, ib0, ia1, ib1, rows_a, rows_b, acc,
                  sa0, sb0, sa1, sb1, sga, sgb, ssa, ssb):
    c = lax.axis_index("c")
    s = lax.axis_index("s")
    wid = s * NC + c
    ibufs = (ia0, ib0, ia1, ib1)
    isems = (sa0, sb0, sa1, sb1)
    rbufs = (rows_a, rows_b)
    gsems = (sga, sgb)

    def _iload(j, p):
        pltpu.async_copy(edg_hbm.at[wid, j], ibufs[p], isems[p])

    def _iwait(p):
        pltpu.make_async_copy(edg_hbm.at[wid, 0], ibufs[p], isems[p]).wait()

    def _g(j, p):
        pltpu.async_copy(hs_hbm.at[ibufs[p].at[0]], rbufs[p % 2],
                         gsems[p % 2])

    def _gw(q):
        pltpu.make_async_copy(hs_hbm.at[ibufs[q].at[0]], rbufs[q],
                              gsems[q]).wait()

    ssems = (ssa, ssb)

    def _s(j, p):
        pltpu.async_copy(rbufs[p % 2], acc.at[ibufs[p].at[1]],
                         ssems[p % 2], add=True)

    def _sw(q):
        pltpu.make_async_copy(rbufs[q], acc.at[ibufs[q].at[1]],
                              ssems[q]).wait()

    for j in range(4):
        _iload(j, j)
    for r in range(16):
        for k in range(D // 16):
            rows_a[r, pl.ds(k * 16, 16)] = jnp.zeros((16,), _F32)
    row0 = s * RPT
    def _zero(i, carry):
        pltpu.sync_copy(rows_a.at[pl.ds(0, 16), :],
                        acc.at[pl.ds(row0 + i * 16, 16), :])
        return carry
    lax.fori_loop(0, RPT // 16, _zero, 0)
    plsc.subcore_barrier()
    _iwait(0)
    _g(0, 0)
    _iwait(1)
    _g(1, 1)

    def _step_loop(t, carry):
        j = 4 * t
        for p in (0, 2):
            _gw(0)
            _s(j + p, p)
            _gw(1)
            _s(j + p + 1, p + 1)
            _sw(0)
            _iload(j + p + 4, p)
            _iwait((p + 2) % 4)
            _g(j + p + 2, (p + 2) % 4)
            _sw(1)
            _iload(j + p + 5, p + 1)
            _iwait((p + 3) % 4)
            _g(j + p + 3, (p + 3) % 4)
        return carry
    lax.fori_loop(0, (NCHUNK - 8) // 4, _step_loop, 0)

    for jj in range(NCHUNK - 8, NCHUNK, 2):
        p = jj % 4
        _gw(0)
        _s(jj, p)
        _gw(1)
        _s(jj + 1, p + 1)
        _sw(0)
        if jj + 4 < NCHUNK:
            _iload(jj + 4, p)
        if jj + 2 < NCHUNK:
            _iwait((p + 2) % 4)
            _g(jj + 2, (p + 2) % 4)
        _sw(1)
        if jj + 5 < NCHUNK:
            _iload(jj + 5, p + 1)
        if jj + 3 < NCHUNK:
            _iwait((p + 3) % 4)
            _g(jj + 3, (p + 3) % 4)
    plsc.subcore_barrier()
    pltpu.sync_copy(acc.at[pl.ds(row0, RPT), :],
                    out_hbm.at[c, pl.ds(row0, RPT), :])


def _dinv_from(degp):
    deg = degp[0, :, 0] + degp[1, :, 0] + 1.0
    return lax.rsqrt(deg)


def _tc_prep_body(x_ref, w_ref, degp_ref, h_ref, hs_ref):
    dinv = _dinv_from(degp_ref[...])
    h = jnp.dot(x_ref[...], w_ref[...], preferred_element_type=_F32,
                precision=lax.Precision.HIGHEST)
    h_ref[...] = h
    hs_ref[...] = h * dinv[:, None]


_tc_prep = pl.pallas_call(
    _tc_prep_body,
    out_shape=[jax.ShapeDtypeStruct((NP, D), _F32),
               jax.ShapeDtypeStruct((NP, D), _F32)],
)


def _tc_combine_body(p_ref, h_ref, degp_ref, b_ref, z_ref, st_ref):
    dinv = _dinv_from(degp_ref[...])
    h = h_ref[...]
    z = ((p_ref[0] + p_ref[1]) * dinv[:, None]
         + h * (dinv * dinv)[:, None] + b_ref[...][None, :])
    z_ref[...] = z
    rows = lax.broadcasted_iota(jnp.int32, (NP, 1), 0)
    mask = (rows < N).astype(_F32)
    mu = jnp.sum(z * mask, axis=0, keepdims=True) / N
    dz = (z - mu) * mask
    var = jnp.sum(dz * dz, axis=0, keepdims=True) / N
    st_ref[...] = jnp.concatenate([mu, var], axis=0)


_tc_combine = pl.pallas_call(
    _tc_combine_body,
    out_shape=[jax.ShapeDtypeStruct((NP, D), _F32),
               jax.ShapeDtypeStruct((2, D), _F32)],
)


def _tc_norm_mm_body(z_ref, st_ref, degp_ref, g_ref, be_ref, w_ref,
                     hn_ref, hsn_ref):
    dinv = _dinv_from(degp_ref[...])
    mu = st_ref[0][None, :]
    var = st_ref[1][None, :]
    zn = (z_ref[...] - mu) * lax.rsqrt(var + 1e-5) * g_ref[...][None, :] \
        + be_ref[...][None, :]
    rows = lax.broadcasted_iota(jnp.int32, (NP, 1), 0)
    mask = (rows < N).astype(_F32)
    a = jnp.maximum(zn, 0.0) * mask
    hn = jnp.dot(a, w_ref[...], preferred_element_type=_F32,
                 precision=lax.Precision.HIGHEST)
    hn_ref[...] = hn
    hsn_ref[...] = hn * dinv[:, None]


_tc_norm_mm = pl.pallas_call(
    _tc_norm_mm_body,
    out_shape=[jax.ShapeDtypeStruct((NP, D), _F32),
               jax.ShapeDtypeStruct((NP, D), _F32)],
)


def _tc_mid(p, h, degp, b, g, be, w):
    z, st = _tc_combine(p, h, degp, b)
    return _tc_norm_mm(z, st, degp, g, be, w)


def _tc_final_body(p_ref, h_ref, degp_ref, b_ref, out_ref):
    dinv = _dinv_from(degp_ref[...])
    h = h_ref[...]
    out_ref[...] = ((p_ref[0] + p_ref[1]) * dinv[:, None]
                    + h * (dinv * dinv)[:, None] + b_ref[...][None, :])


_tc_final = pl.pallas_call(
    _tc_final_body,
    out_shape=jax.ShapeDtypeStruct((NP, D), _F32),
)


def kernel(x, edge_index, W1, b1, g1, be1, W2, b2, g2, be2, W3, b3):
    xp = jnp.concatenate([x, jnp.zeros((NP - N, D), _F32)], axis=0)
    rpw = E // NW
    pads = N + jnp.arange(EPW - rpw, dtype=jnp.int32) % (NP - N)
    padw = jnp.broadcast_to(pads, (NW, EPW - rpw))
    src3 = jnp.concatenate([edge_index[0].reshape(NW, rpw), padw],
                           axis=1).reshape(NW, NCHUNK, CH)
    dst3 = jnp.concatenate([edge_index[1].reshape(NW, rpw), padw],
                           axis=1).reshape(NW, NCHUNK, CH)
    edg = jnp.stack([src3, dst3], axis=2)

    degp = _sc_degree(edg)
    h1, hs1 = _tc_prep(xp, W1, degp)
    p1 = _sc_aggregate(hs1, edg)
    h2, hs2 = _tc_mid(p1, h1, degp, b1, g1, be1, W2)
    p2 = _sc_aggregate(hs2, edg)
    h3, hs3 = _tc_mid(p2, h2, degp, b2, g2, be2, W3)
    p3 = _sc_aggregate(hs3, edg)
    outp = _tc_final(p3, h3, degp, b3)
    return outp[:N]


# speedup vs baseline: 1.3121x; 1.3121x over previous
import functools

import jax
import jax.numpy as jnp
from jax import lax
from jax.experimental import pallas as pl
from jax.experimental.pallas import tpu as pltpu
from jax.experimental.pallas import tpu_sc as plsc

N = 10000
D = 128
E = 320000
NC = 2
NS = 16
NW = NC * NS
CH = 128
NCHUNK = 80
EPW = NCHUNK * CH
EPAD = EPW * NW
NP = 10240
RPT = NP // NS
_F32 = jnp.float32

_MESH = plsc.VectorSubcoreMesh(core_axis_name="c", subcore_axis_name="s")


@functools.partial(
    pl.kernel,
    out_type=jax.ShapeDtypeStruct((NC, NP, 16), _F32),
    mesh=_MESH,
    scratch_types=[
        pltpu.VMEM((NCHUNK, 2, CH), jnp.int32),
        pltpu.VMEM((CH, 16), _F32),
        pltpu.VMEM((16, 16), _F32),
        pltpu.VMEM_SHARED((NP, 16), _F32),
        pltpu.SemaphoreType.DMA,
    ],
)
def _sc_degree(edg_hbm, out_hbm, idx_v, ones_v, zbuf, hist, sem):
    c = lax.axis_index("c")
    s = lax.axis_index("s")
    wid = s * NC + c
    pltpu.async_copy(edg_hbm.at[wid], idx_v, sem)
    for r in range(16):
        zbuf[r] = jnp.zeros((16,), _F32)
    for r in range(CH):
        ones_v[r] = jnp.ones((16,), _F32)
    row0 = s * RPT
    def _zero(i, carry):
        pltpu.sync_copy(zbuf, hist.at[pl.ds(row0 + i * 16, 16), :])
        return carry
    lax.fori_loop(0, RPT // 16, _zero, 0)
    pltpu.make_async_copy(edg_hbm.at[wid], idx_v, sem).wait()
    plsc.subcore_barrier()
    def _acc(j, carry):
        pltpu.sync_copy(ones_v, hist.at[idx_v.at[j, 1]], add=True)
        return carry
    lax.fori_loop(0, NCHUNK, _acc, 0)
    plsc.subcore_barrier()
    pltpu.sync_copy(hist.at[pl.ds(row0, RPT), :],
                    out_hbm.at[c, pl.ds(row0, RPT), :])


@functools.partial(
    pl.kernel,
    out_type=jax.ShapeDtypeStruct((NC, NP, D), _F32),
    mesh=_MESH,
    scratch_types=[
        pltpu.VMEM((2, CH), jnp.int32),
        pltpu.VMEM((2, CH), jnp.int32),
        pltpu.VMEM((2, CH), jnp.int32),
        pltpu.VMEM((2, CH), jnp.int32),
        pltpu.VMEM((CH, D), _F32),
        pltpu.VMEM((CH, D), _F32),
        pltpu.VMEM_SHARED((NP, D), _F32),
        pltpu.SemaphoreType.DMA,
        pltpu.SemaphoreType.DMA,
        pltpu.SemaphoreType.DMA,
        pltpu.SemaphoreType.DMA,
        pltpu.SemaphoreType.DMA,
        pltpu.SemaphoreType.DMA,
    ],
)
def _sc_aggregate(hs_hbm, edg_hbm, out_hbm,
                  ia0, ib0, ia1, ib1, rows_a, rows_b, acc,
                  sa0, sb0, sa1, sb1, sga, sgb):
    c = lax.axis_index("c")
    s = lax.axis_index("s")
    wid = s * NC + c
    ibufs = (ia0, ib0, ia1, ib1)
    isems = (sa0, sb0, sa1, sb1)
    rbufs = (rows_a, rows_b)
    gsems = (sga, sgb)

    def _iload(j, p):
        pltpu.async_copy(edg_hbm.at[wid, j], ibufs[p], isems[p])

    def _iwait(p):
        pltpu.make_async_copy(edg_hbm.at[wid, 0], ibufs[p], isems[p]).wait()

    def _g(j, p):
        pltpu.async_copy(hs_hbm.at[ibufs[p].at[0]], rbufs[p % 2],
                         gsems[p % 2])

    def _gw(q):
        pltpu.make_async_copy(hs_hbm.at[ibufs[q].at[0]], rbufs[q],
                              gsems[q]).wait()

    def _s(j, p):
        pltpu.sync_copy(rbufs[p % 2], acc.at[ibufs[p].at[1]], add=True)

    for j in range(4):
        _iload(j, j)
    for r in range(16):
        for k in range(D // 16):
            rows_a[r, pl.ds(k * 16, 16)] = jnp.zeros((16,), _F32)
    row0 = s * RPT
    def _zero(i, carry):
        pltpu.sync_copy(rows_a.at[pl.ds(0, 16), :],
                        acc.at[pl.ds(row0 + i * 16, 16), :])
        return carry
    lax.fori_loop(0, RPT // 16, _zero, 0)
    plsc.subcore_barrier()
    _iwait(0)
    _g(0, 0)
    _iwait(1)
    _g(1, 1)

    def _step_loop(t, carry):
        j = 4 * t
        for p in range(4):
            _gw(p % 2)
            _s(j + p, p)
            _iload(j + p + 4, p)
            _iwait((p + 2) % 4)
            _g(j + p + 2, (p + 2) % 4)
        return carry
    lax.fori_loop(0, (NCHUNK - 8) // 4, _step_loop, 0)

    for jj in range(NCHUNK - 8, NCHUNK):
        p = jj % 4
        _gw(p % 2)
        _s(jj, p)
        if jj + 4 < NCHUNK:
            _iload(jj + 4, p)
        if jj + 2 < NCHUNK:
            _iwait((p + 2) % 4)
            _g(jj + 2, (p + 2) % 4)
    plsc.subcore_barrier()
    pltpu.sync_copy(acc.at[pl.ds(row0, RPT), :],
                    out_hbm.at[c, pl.ds(row0, RPT), :])


def _dinv_from(degp):
    deg = degp[0, :, 0] + degp[1, :, 0] + 1.0
    return lax.rsqrt(deg)


def _tc_matmul_body(x_ref, w_ref, h_ref):
    h_ref[...] = jnp.dot(x_ref[...], w_ref[...], preferred_element_type=_F32,
                         precision=lax.Precision.HIGHEST)


_tc_matmul = pl.pallas_call(
    _tc_matmul_body,
    out_shape=jax.ShapeDtypeStruct((NP, D), _F32),
)


def _tc_scale_body(h_ref, degp_ref, hs_ref):
    dinv = _dinv_from(degp_ref[...])
    hs_ref[...] = h_ref[...] * dinv[:, None]


_tc_scale = pl.pallas_call(
    _tc_scale_body,
    out_shape=jax.ShapeDtypeStruct((NP, D), _F32),
)


_PADR = float(NP - N)


def _tc_mid_body(p_ref, hs_ref, degp_ref, b_ref, g_ref, be_ref, w_ref,
                 hsn_ref):
    dinv = _dinv_from(degp_ref[...])
    b = b_ref[...][None, :]
    z = (p_ref[0] + p_ref[1] + hs_ref[...]) * dinv[:, None] + b
    sall = jnp.sum(z, axis=0, keepdims=True)
    ssall = jnp.sum(z * z, axis=0, keepdims=True)
    mu = (sall - _PADR * b) / N
    var = (ssall - _PADR * b * b) / N - mu * mu
    zn = (z - mu) * lax.rsqrt(var + 1e-5) * g_ref[...][None, :] \
        + be_ref[...][None, :]
    rows = lax.broadcasted_iota(jnp.int32, (NP, 1), 0)
    mask = (rows < N).astype(_F32)
    a = jnp.maximum(zn, 0.0) * mask
    hn = jnp.dot(a, w_ref[...], preferred_element_type=_F32,
                 precision=lax.Precision.HIGHEST)
    hsn_ref[...] = hn * dinv[:, None]


_tc_mid = pl.pallas_call(
    _tc_mid_body,
    out_shape=jax.ShapeDtypeStruct((NP, D), _F32),
)


def _tc_final_body(p_ref, hs_ref, degp_ref, b_ref, out_ref):
    dinv = _dinv_from(degp_ref[...])
    out_ref[...] = ((p_ref[0] + p_ref[1] + hs_ref[...]) * dinv[:, None]
                    + b_ref[...][None, :])


_tc_final = pl.pallas_call(
    _tc_final_body,
    out_shape=jax.ShapeDtypeStruct((NP, D), _F32),
)


def kernel(x, edge_index, W1, b1, g1, be1, W2, b2, g2, be2, W3, b3):
    xp = jnp.concatenate([x, jnp.zeros((NP - N, D), _F32)], axis=0)
    rpw = E // NW
    pads = N + jnp.arange(EPW - rpw, dtype=jnp.int32) % (NP - N)
    padw = jnp.broadcast_to(pads, (NW, EPW - rpw))
    src3 = jnp.concatenate([edge_index[0].reshape(NW, rpw), padw],
                           axis=1).reshape(NW, NCHUNK, CH)
    dst3 = jnp.concatenate([edge_index[1].reshape(NW, rpw), padw],
                           axis=1).reshape(NW, NCHUNK, CH)
    edg = jnp.stack([src3, dst3], axis=2)

    degp = _sc_degree(edg)
    h1 = _tc_matmul(xp, W1)
    hs1 = _tc_scale(h1, degp)
    p1 = _sc_aggregate(hs1, edg)
    hs2 = _tc_mid(p1, hs1, degp, b1, g1, be1, W2)
    p2 = _sc_aggregate(hs2, edg)
    hs3 = _tc_mid(p2, hs2, degp, b2, g2, be2, W3)
    p3 = _sc_aggregate(hs3, edg)
    outp = _tc_final(p3, hs3, degp, b3)
    return outp[:N]
